# baseline (speedup 1.0000x reference)
import functools

import jax
import jax.numpy as jnp
import numpy as np
from jax import lax
from jax.experimental import pallas as pl
from jax.experimental.pallas import tpu as pltpu
from jax.experimental.pallas import tpu_sc as plsc

B, N, D, NH = 8, 10000, 128, 16
TOTAL = B * N
LANES = 16
NW = 32
CHUNK_NODES = 16
CHUNK_IDX = CHUNK_NODES * NH
IDX_PER_XFER = 128
XFERS = CHUNK_IDX // IDX_PER_XFER
N_CHUNKS = TOTAL // CHUNK_NODES
STEPS = -(-N_CHUNKS // NW)
PAIRS = (STEPS + 2) // 2

MM_BLK = 640

_PERM = np.empty(D, dtype=np.int32)
for _c in range(D // 32):
    for _i in range(16):
        _PERM[32 * _c + 2 * _i] = 32 * _c + _i
        _PERM[32 * _c + 2 * _i + 1] = 32 * _c + 16 + _i


def _mm_body(x_ref, w_ref, o_ref):
    o_ref[...] = jnp.dot(x_ref[...], w_ref[...] * (1.0 / NH),
                         preferred_element_type=jnp.float32
                         ).astype(jnp.bfloat16)


def _matmul(x2, w):
    return pl.pallas_call(
        _mm_body,
        grid=(TOTAL // MM_BLK,),
        in_specs=[pl.BlockSpec((MM_BLK, D), lambda i: (i, 0)),
                  pl.BlockSpec((D, D), lambda i: (0, 0))],
        out_specs=pl.BlockSpec((MM_BLK, D), lambda i: (i, 0)),
        out_shape=jax.ShapeDtypeStruct((TOTAL, D), jnp.bfloat16),
    )(x2, w)


def _sc_gather_sum(y2, gidx):
    mesh = plsc.VectorSubcoreMesh(core_axis_name="c", subcore_axis_name="s")

    @functools.partial(
        pl.kernel,
        mesh=mesh,
        compiler_params=pltpu.CompilerParams(use_tc_tiling_on_sc=False),
        out_type=jax.ShapeDtypeStruct((TOTAL, D), jnp.float32),
        scratch_types=[
            pltpu.VMEM((CHUNK_IDX,), jnp.int32),
            pltpu.VMEM((CHUNK_IDX,), jnp.int32),
            pltpu.VMEM((CHUNK_IDX, D // 2), jnp.int32),
            pltpu.VMEM((CHUNK_IDX, D // 2), jnp.int32),
            pltpu.VMEM((CHUNK_NODES, D), jnp.float32),
            pltpu.SemaphoreType.DMA,
            pltpu.SemaphoreType.DMA,
        ],
    )
    def k(y_hbm, gidx_hbm, out_hbm, idx0, idx1, rows0, rows1, out_v,
          sem0, sem1):
        wid = lax.axis_index("s") * 2 + lax.axis_index("c")
        idx_bufs = (idx0, idx1)
        row_bufs = (rows0, rows1)
        sems = (sem0, sem1)

        def gather_copies(idx_v, rows_v, sem):
            return [pltpu.make_async_copy(
                        y_hbm.at[idx_v.at[pl.ds(j * IDX_PER_XFER,
                                                IDX_PER_XFER)]],
                        rows_v.at[pl.ds(j * IDX_PER_XFER, IDX_PER_XFER)],
                        sem)
                    for j in range(XFERS)]

        def fire(t, p):
            cid = wid + t * NW

            @pl.when(cid < N_CHUNKS)
            def _():
                idx_v = idx_bufs[p]
                node0 = cid * CHUNK_NODES
                off_vec = jnp.full((LANES,), (node0 // N) * N,
                                   dtype=jnp.int32)
                pltpu.sync_copy(
                    gidx_hbm.at[pl.ds(cid * CHUNK_IDX, CHUNK_IDX)], idx_v)
                for i in range(CHUNK_IDX // LANES):
                    sl = pl.ds(i * LANES, LANES)
                    idx_v[sl] = idx_v[sl] + off_vec
                for cp in gather_copies(idx_v, row_bufs[p], sems[p]):
                    cp.start()

        def consume(t, p):
            cid = wid + t * NW

            @pl.when(cid < N_CHUNKS)
            def _():
                rows_v = row_bufs[p]
                for cp in gather_copies(idx_bufs[p], rows_v, sems[p]):
                    cp.wait()

                hi_mask = jnp.full((LANES,), -65536, dtype=jnp.int32)

                def bf16_pair(r, c):
                    v = rows_v[r, pl.ds(c * LANES, LANES)]
                    a = lax.bitcast_convert_type(v << 16, jnp.float32)
                    b = lax.bitcast_convert_type(v & hi_mask, jnp.float32)
                    return a, b

                def node_body(m, _):
                    r0 = m * NH
                    for c in range(D // 32):
                        acc_a, acc_b = bf16_pair(r0, c)
                        for r in range(1, NH):
                            a, b = bf16_pair(r0 + r, c)
                            acc_a = acc_a + a
                            acc_b = acc_b + b
                        out_v[m, pl.ds(32 * c, LANES)] = acc_a
                        out_v[m, pl.ds(32 * c + LANES, LANES)] = acc_b
                    return 0

                lax.fori_loop(0, CHUNK_NODES, node_body, 0)
                pltpu.sync_copy(out_v,
                                out_hbm.at[pl.ds(cid * CHUNK_NODES,
                                                 CHUNK_NODES)])

        fire(0, 0)

        def pair_body(kk, _):
            t = 2 * kk
            fire(t + 1, 1)
            consume(t, 0)
            fire(t + 2, 0)
            consume(t + 1, 1)
            return 0

        lax.fori_loop(0, PAIRS, pair_body, 0)

    return k(y2, gidx)


def kernel(x, local_cell_indices_nh, W):
    x2 = x.reshape(TOTAL, D)
    y2 = _matmul(x2, W[:, _PERM])
    y2i = lax.bitcast_convert_type(y2.reshape(TOTAL, D // 2, 2), jnp.int32)
    gidx = local_cell_indices_nh.astype(jnp.int32).reshape(TOTAL * NH)
    out2 = _sc_gather_sum(y2i, gidx)
    return out2.reshape(B, N, D)


# speedup vs baseline: 1.5454x; 1.5454x over previous
import functools

import jax
import jax.numpy as jnp
import numpy as np
from jax import lax
from jax.experimental import pallas as pl
from jax.experimental.pallas import tpu as pltpu
from jax.experimental.pallas import tpu_sc as plsc

B, N, D, NH = 8, 10000, 128, 16
TOTAL = B * N
LANES = 16
NW = 32
CHUNK_NODES = 32
CHUNK_IDX = CHUNK_NODES * NH
IDX_PER_XFER = 128
XFERS = CHUNK_IDX // IDX_PER_XFER
N_CHUNKS = TOTAL // CHUNK_NODES
STEPS = -(-N_CHUNKS // NW)
PAIRS = (STEPS + 2) // 2
WORDS = D // 2

MM_BLK = 640

_PERM = np.empty(D, dtype=np.int32)
for _c in range(D // 32):
    for _i in range(16):
        _PERM[16 * _c + _i] = 32 * _c + _i
        _PERM[64 + 16 * _c + _i] = 32 * _c + 16 + _i


def _mm_body(x_ref, w_ref, o_ref):
    yf = jnp.dot(x_ref[...], w_ref[...] * (1.0 / NH),
                 preferred_element_type=jnp.float32)
    lo = lax.bitcast_convert_type(yf[:, :WORDS].astype(jnp.bfloat16),
                                  jnp.uint16).astype(jnp.uint32)
    hi = lax.bitcast_convert_type(yf[:, WORDS:].astype(jnp.bfloat16),
                                  jnp.uint16).astype(jnp.uint32)
    o_ref[...] = lax.bitcast_convert_type(lo | (hi << 16), jnp.int32)


def _matmul(x2, w):
    return pl.pallas_call(
        _mm_body,
        grid=(TOTAL // MM_BLK,),
        in_specs=[pl.BlockSpec((MM_BLK, D), lambda i: (i, 0)),
                  pl.BlockSpec((D, D), lambda i: (0, 0))],
        out_specs=pl.BlockSpec((MM_BLK, WORDS), lambda i: (i, 0)),
        out_shape=jax.ShapeDtypeStruct((TOTAL, WORDS), jnp.int32),
    )(x2, w)


def _sc_gather_sum(y2i, gidx):
    mesh = plsc.VectorSubcoreMesh(core_axis_name="c", subcore_axis_name="s")

    @functools.partial(
        pl.kernel,
        mesh=mesh,
        compiler_params=pltpu.CompilerParams(use_tc_tiling_on_sc=False),
        out_type=jax.ShapeDtypeStruct((TOTAL, D), jnp.float32),
        scratch_types=[
            pltpu.VMEM((CHUNK_IDX,), jnp.int32),
            pltpu.VMEM((CHUNK_IDX,), jnp.int32),
            pltpu.VMEM((CHUNK_IDX, WORDS), jnp.int32),
            pltpu.VMEM((CHUNK_IDX, WORDS), jnp.int32),
            pltpu.VMEM((CHUNK_NODES, D), jnp.float32),
            pltpu.VMEM((CHUNK_NODES, D), jnp.float32),
            pltpu.SemaphoreType.DMA,
            pltpu.SemaphoreType.DMA,
            pltpu.SemaphoreType.DMA,
            pltpu.SemaphoreType.DMA,
        ],
    )
    def k(y_hbm, gidx_hbm, out_hbm, idx0, idx1, rows0, rows1, out0, out1,
          sem0, sem1, osem0, osem1):
        wid = lax.axis_index("s") * 2 + lax.axis_index("c")
        idx_bufs = (idx0, idx1)
        row_bufs = (rows0, rows1)
        out_bufs = (out0, out1)
        sems = (sem0, sem1)
        osems = (osem0, osem1)

        def gather_copies(idx_v, rows_v, sem):
            return [pltpu.make_async_copy(
                        y_hbm.at[idx_v.at[pl.ds(j * IDX_PER_XFER,
                                                IDX_PER_XFER)]],
                        rows_v.at[pl.ds(j * IDX_PER_XFER, IDX_PER_XFER)],
                        sem)
                    for j in range(XFERS)]

        def out_copy(t, p):
            cid = wid + t * NW
            return pltpu.make_async_copy(
                out_bufs[p],
                out_hbm.at[pl.ds(cid * CHUNK_NODES, CHUNK_NODES)],
                osems[p])

        def fire(t, p):
            cid = wid + t * NW

            @pl.when(cid < N_CHUNKS)
            def _():
                idx_v = idx_bufs[p]
                node0 = cid * CHUNK_NODES
                pltpu.sync_copy(
                    gidx_hbm.at[pl.ds(cid * CHUNK_IDX, CHUNK_IDX)], idx_v)
                for g in range(CHUNK_NODES // 16):
                    off_vec = jnp.full(
                        (LANES,), ((node0 + 16 * g) // N) * N,
                        dtype=jnp.int32)
                    for i in range(16 * NH // LANES):
                        sl = pl.ds(g * 16 * NH + i * LANES, LANES)
                        idx_v[sl] = idx_v[sl] + off_vec
                for cp in gather_copies(idx_v, row_bufs[p], sems[p]):
                    cp.start()

        def consume(t, p):
            cid = wid + t * NW

            @pl.when(cid < N_CHUNKS)
            def _():
                rows_v = row_bufs[p]
                out_v = out_bufs[p]
                for cp in gather_copies(idx_bufs[p], rows_v, sems[p]):
                    cp.wait()

                @pl.when(cid >= 2 * NW)
                def _():
                    out_copy(t - 2, p).wait()

                hi_mask = jnp.full((LANES,), -65536, dtype=jnp.int32)

                def bf16_pair(r, c):
                    v = rows_v[r, pl.ds(c * LANES, LANES)]
                    a = lax.bitcast_convert_type(v << 16, jnp.float32)
                    b = lax.bitcast_convert_type(v & hi_mask, jnp.float32)
                    return a, b

                def node_body(m, _):
                    r0 = m * NH
                    for c in range(D // 32):
                        acc_a, acc_b = bf16_pair(r0, c)
                        for r in range(1, NH):
                            a, b = bf16_pair(r0 + r, c)
                            acc_a = acc_a + a
                            acc_b = acc_b + b
                        out_v[m, pl.ds(32 * c, LANES)] = acc_a
                        out_v[m, pl.ds(32 * c + LANES, LANES)] = acc_b
                    return 0

                lax.fori_loop(0, CHUNK_NODES, node_body, 0)
                out_copy(t, p).start()

        fire(0, 0)

        def pair_body(kk, _):
            t = 2 * kk
            fire(t + 1, 1)
            consume(t, 0)
            fire(t + 2, 0)
            consume(t + 1, 1)
            return 0

        lax.fori_loop(0, PAIRS, pair_body, 0)

        nv = (N_CHUNKS - wid + NW - 1) // NW
        for p in (0, 1):
            @pl.when((nv >= 1) & ((nv - 1) % 2 == p))
            def _(p=p):
                out_copy(nv - 1, p).wait()

            @pl.when((nv >= 2) & ((nv - 2) % 2 == p))
            def _(p=p):
                out_copy(nv - 2, p).wait()

    return k(y2i, gidx)


def kernel(x, local_cell_indices_nh, W):
    x2 = x.reshape(TOTAL, D)
    y2i = _matmul(x2, W[:, _PERM])
    gidx = local_cell_indices_nh.astype(jnp.int32).reshape(TOTAL * NH)
    out2 = _sc_gather_sum(y2i, gidx)
    return out2.reshape(B, N, D)


# speedup vs baseline: 1.7137x; 1.1089x over previous
import functools

import jax
import jax.numpy as jnp
import numpy as np
from jax import lax
from jax.experimental import pallas as pl
from jax.experimental.pallas import tpu as pltpu
from jax.experimental.pallas import tpu_sc as plsc

B, N, D, NH = 8, 10000, 128, 16
TOTAL = B * N
LANES = 16
NW = 32
CHUNK_NODES = 32
CHUNK_IDX = CHUNK_NODES * NH
IDX_PER_XFER = 128
XFERS = CHUNK_IDX // IDX_PER_XFER
N_CHUNKS = TOTAL // CHUNK_NODES
STEPS = -(-N_CHUNKS // NW)
PAIRS = (STEPS + 2) // 2
WORDS = D // 2

MM_BLK = 2000

_PERM = np.empty(D, dtype=np.int32)
for _c in range(D // 32):
    for _i in range(16):
        _PERM[16 * _c + _i] = 32 * _c + _i
        _PERM[64 + 16 * _c + _i] = 32 * _c + 16 + _i


def _mm_body(x_ref, w_ref, o_ref):
    yf = jnp.dot(x_ref[...], w_ref[...] * (1.0 / NH),
                 preferred_element_type=jnp.float32)
    lo = lax.bitcast_convert_type(yf[:, :WORDS].astype(jnp.bfloat16),
                                  jnp.uint16).astype(jnp.uint32)
    hi = lax.bitcast_convert_type(yf[:, WORDS:].astype(jnp.bfloat16),
                                  jnp.uint16).astype(jnp.uint32)
    o_ref[...] = lax.bitcast_convert_type(lo | (hi << 16), jnp.int32)


def _matmul(x2, w):
    return pl.pallas_call(
        _mm_body,
        grid=(TOTAL // MM_BLK,),
        in_specs=[pl.BlockSpec((MM_BLK, D), lambda i: (i, 0)),
                  pl.BlockSpec((D, D), lambda i: (0, 0))],
        out_specs=pl.BlockSpec((MM_BLK, WORDS), lambda i: (i, 0)),
        out_shape=jax.ShapeDtypeStruct((TOTAL, WORDS), jnp.int32),
    )(x2, w)


def _sc_gather_sum(y2i, gidx):
    mesh = plsc.VectorSubcoreMesh(core_axis_name="c", subcore_axis_name="s")

    @functools.partial(
        pl.kernel,
        mesh=mesh,
        compiler_params=pltpu.CompilerParams(use_tc_tiling_on_sc=False),
        out_type=jax.ShapeDtypeStruct((TOTAL, D), jnp.float32),
        scratch_types=[
            pltpu.VMEM((CHUNK_IDX,), jnp.int32),
            pltpu.VMEM((CHUNK_IDX,), jnp.int32),
            pltpu.VMEM((CHUNK_IDX, WORDS), jnp.int32),
            pltpu.VMEM((CHUNK_IDX, WORDS), jnp.int32),
            pltpu.VMEM((CHUNK_NODES, D), jnp.float32),
            pltpu.VMEM((CHUNK_NODES, D), jnp.float32),
            pltpu.SemaphoreType.DMA,
            pltpu.SemaphoreType.DMA,
            pltpu.SemaphoreType.DMA,
            pltpu.SemaphoreType.DMA,
        ],
    )
    def k(y_hbm, gidx_hbm, out_hbm, idx0, idx1, rows0, rows1, out0, out1,
          sem0, sem1, osem0, osem1):
        wid = lax.axis_index("s") * 2 + lax.axis_index("c")
        idx_bufs = (idx0, idx1)
        row_bufs = (rows0, rows1)
        out_bufs = (out0, out1)
        sems = (sem0, sem1)
        osems = (osem0, osem1)

        def gather_copies(idx_v, rows_v, sem):
            return [pltpu.make_async_copy(
                        y_hbm.at[idx_v.at[pl.ds(j * IDX_PER_XFER,
                                                IDX_PER_XFER)]],
                        rows_v.at[pl.ds(j * IDX_PER_XFER, IDX_PER_XFER)],
                        sem)
                    for j in range(XFERS)]

        def out_copy(t, p):
            cid = wid + t * NW
            return pltpu.make_async_copy(
                out_bufs[p],
                out_hbm.at[pl.ds(cid * CHUNK_NODES, CHUNK_NODES)],
                osems[p])

        def fire(t, p):
            cid = wid + t * NW

            @pl.when(cid < N_CHUNKS)
            def _():
                idx_v = idx_bufs[p]
                node0 = cid * CHUNK_NODES
                pltpu.sync_copy(
                    gidx_hbm.at[pl.ds(cid * CHUNK_IDX, CHUNK_IDX)], idx_v)
                for g in range(CHUNK_NODES // 16):
                    off_vec = jnp.full(
                        (LANES,), ((node0 + 16 * g) // N) * N,
                        dtype=jnp.int32)
                    for i in range(16 * NH // LANES):
                        sl = pl.ds(g * 16 * NH + i * LANES, LANES)
                        idx_v[sl] = idx_v[sl] + off_vec
                for cp in gather_copies(idx_v, row_bufs[p], sems[p]):
                    cp.start()

        def consume(t, p):
            cid = wid + t * NW

            @pl.when(cid < N_CHUNKS)
            def _():
                rows_v = row_bufs[p]
                out_v = out_bufs[p]
                for cp in gather_copies(idx_bufs[p], rows_v, sems[p]):
                    cp.wait()

                @pl.when(cid >= 2 * NW)
                def _():
                    out_copy(t - 2, p).wait()

                hi_mask = jnp.full((LANES,), -65536, dtype=jnp.int32)

                def bf16_pair(r, c):
                    v = rows_v[r, pl.ds(c * LANES, LANES)]
                    a = lax.bitcast_convert_type(v << 16, jnp.float32)
                    b = lax.bitcast_convert_type(v & hi_mask, jnp.float32)
                    return a, b

                def node_body(m, _):
                    r0 = m * NH
                    for c in range(D // 32):
                        acc_a, acc_b = bf16_pair(r0, c)
                        for r in range(1, NH):
                            a, b = bf16_pair(r0 + r, c)
                            acc_a = acc_a + a
                            acc_b = acc_b + b
                        out_v[m, pl.ds(32 * c, LANES)] = acc_a
                        out_v[m, pl.ds(32 * c + LANES, LANES)] = acc_b
                    return 0

                lax.fori_loop(0, CHUNK_NODES, node_body, 0)
                out_copy(t, p).start()

        fire(0, 0)

        def pair_body(kk, _):
            t = 2 * kk
            fire(t + 1, 1)
            consume(t, 0)
            fire(t + 2, 0)
            consume(t + 1, 1)
            return 0

        lax.fori_loop(0, PAIRS, pair_body, 0)

        nv = (N_CHUNKS - wid + NW - 1) // NW
        for p in (0, 1):
            @pl.when((nv >= 1) & ((nv - 1) % 2 == p))
            def _(p=p):
                out_copy(nv - 1, p).wait()

            @pl.when((nv >= 2) & ((nv - 2) % 2 == p))
            def _(p=p):
                out_copy(nv - 2, p).wait()

    return k(y2i, gidx)


def kernel(x, local_cell_indices_nh, W):
    x2 = x.reshape(TOTAL, D)
    y2i = _matmul(x2, W[:, _PERM])
    gidx = local_cell_indices_nh.astype(jnp.int32).reshape(TOTAL * NH)
    out2 = _sc_gather_sum(y2i, gidx)
    return out2.reshape(B, N, D)
